# baseline (speedup 1.0000x reference)
import functools

import jax
import jax.numpy as jnp
from jax import lax
from jax.experimental import pallas as pl
from jax.experimental.pallas import tpu as pltpu
from jax.experimental.pallas import tpu_sc as plsc

D = 128
NC = 2
NS = 16
NW = NC * NS
G = 128


NBUF = 4


def _sc_gather(table, idx):
    B = idx.shape[0] // G
    idx3 = idx.reshape(B, 1, G)
    nb_w = -(-B // NW)
    mesh = plsc.VectorSubcoreMesh(core_axis_name="c", subcore_axis_name="s")

    @functools.partial(
        pl.kernel,
        mesh=mesh,
        out_type=jax.ShapeDtypeStruct((B, G, D), jnp.float32),
        scratch_types=[
            pltpu.VMEM((NBUF, G), jnp.int32),
            pltpu.VMEM((NBUF, G, D), jnp.float32),
            pltpu.SemaphoreType.DMA((NBUF,)),
            pltpu.SemaphoreType.DMA((NBUF,)),
            pltpu.SemaphoreType.DMA((NBUF,)),
        ],
    )
    def k(table_hbm, idx_hbm, out_hbm, idx_v, rows_v, sem_i, sem_g, sem_o):
        c = lax.axis_index("c")
        s = lax.axis_index("s")
        wid = s * NC + c
        lo = wid * nb_w
        hi = jnp.minimum(lo + nb_w, B)
        n = hi - lo

        def prime(kk, carry):
            pltpu.async_copy(idx_hbm.at[lo + kk, 0], idx_v.at[kk],
                             sem_i.at[kk])
            return carry

        lax.fori_loop(0, jnp.minimum(n, NBUF), prime, 0)

        def body(b, carry):
            j = lax.rem(b - lo, NBUF)
            pltpu.make_async_copy(idx_hbm.at[b, 0], idx_v.at[j],
                                  sem_i.at[j]).wait()

            @pl.when(b - lo >= NBUF)
            def _():
                pltpu.make_async_copy(rows_v.at[j], out_hbm.at[b - NBUF],
                                      sem_o.at[j]).wait()

            pltpu.async_copy(table_hbm.at[idx_v.at[j]], rows_v.at[j],
                             sem_g.at[j]).wait()
            pltpu.async_copy(rows_v.at[j], out_hbm.at[b], sem_o.at[j])

            @pl.when(b + NBUF < hi)
            def _():
                pltpu.async_copy(idx_hbm.at[b + NBUF, 0], idx_v.at[j],
                                 sem_i.at[j])

            return carry

        lax.fori_loop(lo, hi, body, 0)

        def drain(kk, carry):
            b = hi - jnp.minimum(n, NBUF) + kk
            j = lax.rem(b - lo, NBUF)
            pltpu.make_async_copy(rows_v.at[j], out_hbm.at[b],
                                  sem_o.at[j]).wait()
            return carry

        lax.fori_loop(0, jnp.minimum(n, NBUF), drain, 0)

    return k(table, idx3).reshape(B * G, D)


NPAD = 10240


def _sc_segsum(vals, dst3, n_nodes):
    del n_nodes
    B = vals.shape[0] // G
    vals3 = vals.reshape(B, G, D)
    half = B // NC
    nb_s = -(-half // NS)
    rpt = NPAD // NS
    ZR = 64
    nz = rpt // ZR
    SBUF = 2
    mesh = plsc.VectorSubcoreMesh(core_axis_name="c", subcore_axis_name="s")

    @functools.partial(
        pl.kernel,
        mesh=mesh,
        out_type=jax.ShapeDtypeStruct((NC, NPAD, D), jnp.float32),
        scratch_types=[
            pltpu.VMEM((SBUF, G), jnp.int32),
            pltpu.VMEM((SBUF, G, D), jnp.float32),
            pltpu.VMEM((ZR, D), jnp.float32),
            pltpu.VMEM_SHARED((NPAD, D), jnp.float32),
            pltpu.SemaphoreType.DMA((SBUF,)),
            pltpu.SemaphoreType.DMA((SBUF,)),
            pltpu.SemaphoreType.DMA((SBUF,)),
        ],
    )
    def k(vals_hbm, dst_hbm, out_hbm, idx_v, vbuf, zbuf, acc_sh,
          sem_i, sem_v, sem_s):
        c = lax.axis_index("c")
        s = lax.axis_index("s")

        def zfill(i, carry):
            zbuf[i // 8, pl.ds((i % 8) * 16, 16)] = jnp.zeros((16,), jnp.float32)
            return carry

        lax.fori_loop(0, ZR * 8, zfill, 0)

        def zcopy(i, carry):
            pltpu.sync_copy(zbuf, acc_sh.at[pl.ds(s * rpt + i * ZR, ZR)])
            return carry

        lax.fori_loop(0, nz, zcopy, 0)
        plsc.subcore_barrier()

        lo = c * half + s * nb_s
        hi = jnp.minimum(lo + nb_s, (c + 1) * half)
        n = hi - lo

        def prime(kk, carry):
            pltpu.async_copy(dst_hbm.at[lo + kk, 0], idx_v.at[kk],
                             sem_i.at[kk])
            pltpu.async_copy(vals_hbm.at[lo + kk], vbuf.at[kk], sem_v.at[kk])
            return carry

        lax.fori_loop(0, jnp.minimum(n, SBUF), prime, 0)

        def body(b, carry):
            j = lax.rem(b - lo, SBUF)
            pltpu.make_async_copy(dst_hbm.at[b, 0], idx_v.at[j],
                                  sem_i.at[j]).wait()
            pltpu.make_async_copy(vals_hbm.at[b], vbuf.at[j],
                                  sem_v.at[j]).wait()
            pltpu.async_copy(vbuf.at[j], acc_sh.at[idx_v.at[j]], sem_s.at[j],
                             add=True).wait()

            @pl.when(b + SBUF < hi)
            def _():
                pltpu.async_copy(dst_hbm.at[b + SBUF, 0], idx_v.at[j],
                                 sem_i.at[j])
                pltpu.async_copy(vals_hbm.at[b + SBUF], vbuf.at[j],
                                 sem_v.at[j])

            return carry

        lax.fori_loop(lo, hi, body, 0)
        plsc.subcore_barrier()

        def dump(i, carry):
            pltpu.sync_copy(acc_sh.at[pl.ds(s * rpt + i * ZR, ZR)], zbuf)
            pltpu.sync_copy(zbuf, out_hbm.at[c, pl.ds(s * rpt + i * ZR, ZR)])
            return carry

        lax.fori_loop(0, nz, dump, 0)

    return k(vals3, dst3)


def _tc_matmul(xx, w, blk):
    M, K = xx.shape

    def body(x_ref, w_ref, o_ref):
        o_ref[...] = jnp.dot(x_ref[...], w_ref[...],
                             preferred_element_type=jnp.float32)

    return pl.pallas_call(
        body,
        grid=(M // blk,),
        in_specs=[
            pl.BlockSpec((blk, K), lambda i: (i, 0)),
            pl.BlockSpec((K, D), lambda i: (0, 0)),
        ],
        out_specs=pl.BlockSpec((blk, D), lambda i: (i, 0)),
        out_shape=jax.ShapeDtypeStruct((M, D), jnp.float32),
    )(xx, w)


def _tc_base(both, ea, W2, b1, W3, blk):
    E = ea.shape[0]
    DE = ea.shape[1]
    nblk = E // blk
    b1r = b1.reshape(1, D)

    def body(f1_ref, ea_ref, a3_ref, w2_ref, b1_ref, w3_ref, base_ref, q_ref):
        base = (f1_ref[...] + a3_ref[...] + b1_ref[...]
                + jnp.dot(ea_ref[...], w2_ref[...],
                          preferred_element_type=jnp.float32))
        base_ref[...] = base
        q_ref[...] = jnp.dot(jnp.maximum(base, 0.0), w3_ref[...],
                             preferred_element_type=jnp.float32)

    return pl.pallas_call(
        body,
        grid=(nblk,),
        in_specs=[
            pl.BlockSpec((blk, D), lambda i: (i, 0)),
            pl.BlockSpec((blk, DE), lambda i: (i, 0)),
            pl.BlockSpec((blk, D), lambda i: (i + nblk, 0)),
            pl.BlockSpec((DE, D), lambda i: (0, 0)),
            pl.BlockSpec((1, D), lambda i: (0, 0)),
            pl.BlockSpec((D, D), lambda i: (0, 0)),
        ],
        out_specs=[
            pl.BlockSpec((blk, D), lambda i: (i, 0)),
            pl.BlockSpec((blk, D), lambda i: (i, 0)),
        ],
        out_shape=[
            jax.ShapeDtypeStruct((E, D), jnp.float32),
            jax.ShapeDtypeStruct((E, D), jnp.float32),
        ],
    )(both, ea, both, W2, b1r, W3)


def _tc_iter(base, g, q, W3, blk, last):
    E, _ = base.shape

    def body(base_ref, g_ref, q_ref, w3_ref, o_ref):
        qb = q_ref[...]
        up = jnp.roll(qb, -1, axis=0)
        dn = jnp.roll(qb, 1, axis=0)
        even = (lax.broadcasted_iota(jnp.int32, (blk, D), 0) % 2) == 0
        qrev = jnp.where(even, up, dn)
        msg = jnp.maximum(base_ref[...] + g_ref[...] - qrev, 0.0)
        if last:
            o_ref[...] = msg
        else:
            o_ref[...] = jnp.dot(msg, w3_ref[...],
                                 preferred_element_type=jnp.float32)

    return pl.pallas_call(
        body,
        grid=(E // blk,),
        in_specs=[
            pl.BlockSpec((blk, D), lambda i: (i, 0)),
            pl.BlockSpec((blk, D), lambda i: (i, 0)),
            pl.BlockSpec((blk, D), lambda i: (i, 0)),
            pl.BlockSpec((D, D), lambda i: (0, 0)),
        ],
        out_specs=pl.BlockSpec((blk, D), lambda i: (i, 0)),
        out_shape=jax.ShapeDtypeStruct((E, D), jnp.float32),
    )(base, g, q, W3)


def _tc_hsum(part, blk):
    _, n, _ = part.shape

    def body(p_ref, o_ref):
        o_ref[...] = p_ref[0] + p_ref[1]

    return pl.pallas_call(
        body,
        grid=(n // blk,),
        in_specs=[pl.BlockSpec((NC, blk, D), lambda i: (0, i, 0))],
        out_specs=pl.BlockSpec((blk, D), lambda i: (i, 0)),
        out_shape=jax.ShapeDtypeStruct((n, D), jnp.float32),
    )(part)


def _tc_out(x, nf, W4a, W4b, b2, blk):
    n, _ = x.shape
    b2r = b2.reshape(1, D)

    def body(x_ref, nf_ref, wa_ref, wb_ref, b2_ref, o_ref):
        acc = (jnp.dot(x_ref[...], wa_ref[...],
                       preferred_element_type=jnp.float32)
               + jnp.dot(nf_ref[...], wb_ref[...],
                         preferred_element_type=jnp.float32)
               + b2_ref[...])
        o_ref[...] = jnp.maximum(acc, 0.0)

    return pl.pallas_call(
        body,
        grid=(n // blk,),
        in_specs=[
            pl.BlockSpec((blk, D), lambda i: (i, 0)),
            pl.BlockSpec((blk, D), lambda i: (i, 0)),
            pl.BlockSpec((D, D), lambda i: (0, 0)),
            pl.BlockSpec((D, D), lambda i: (0, 0)),
            pl.BlockSpec((1, D), lambda i: (0, 0)),
        ],
        out_specs=pl.BlockSpec((blk, D), lambda i: (i, 0)),
        out_shape=jax.ShapeDtypeStruct((n, D), jnp.float32),
    )(x, nf, W4a, W4b, b2r)


def kernel(x, edge_index, edge_attr, tree_msg, src_eid, tgt_eid,
           W1, b1, W2, W3, W4, b2):
    n_nodes = x.shape[0]
    E = edge_attr.shape[0]
    ET = tree_msg.shape[0]
    n_iters = 4

    src = edge_index[0].astype(jnp.int32)
    dst = edge_index[1].astype(jnp.int32)
    dst3 = dst.reshape(E // G, 1, G)
    inv = jnp.full((E,), ET, jnp.int32).at[tgt_eid].set(src_eid.astype(jnp.int32))

    xw = _tc_matmul(x, W1, blk=2000)
    tw = _tc_matmul(tree_msg, W3, blk=2000)
    tab = jnp.concatenate([xw, tw, jnp.zeros((1, D), jnp.float32)], axis=0)
    both = _sc_gather(tab, jnp.concatenate([src, n_nodes + inv]))
    base, q = _tc_base(both, edge_attr, W2, b1, W3, blk=2560)

    for it in range(n_iters - 1):
        part = _sc_segsum(q, dst3, n_nodes)
        h = _tc_hsum(part, blk=1280)
        g = _sc_gather(h, src)
        q = _tc_iter(base, g, q, W3, blk=2560, last=(it == n_iters - 2))

    part = _sc_segsum(q, dst3, n_nodes)
    nf = _tc_hsum(part, blk=1280)
    return _tc_out(x, nf, W4[:D], W4[D:], b2, blk=2000)
